# baseline (speedup 1.0000x reference)
import functools

import jax
import jax.numpy as jnp
from jax.experimental import pallas as pl
from jax.experimental.pallas import tpu as pltpu

N = 10000
H = 64
D_IN = 128
EMB = 32

_HIGH = jax.lax.Precision.HIGHEST


def _dot(a, b):
    return jax.lax.dot(a, b, precision=_HIGH, preferred_element_type=jnp.float32)


def _matmul_bias_kernel(x_ref, w_ref, b_ref, o_ref):
    o_ref[...] = _dot(x_ref[...], w_ref[...]) + b_ref[...]


def _matmul_bias(x, w, b, block=1000):
    m, k = x.shape
    n = w.shape[1]
    return pl.pallas_call(
        _matmul_bias_kernel,
        grid=(m // block,),
        in_specs=[
            pl.BlockSpec((block, k), lambda i: (i, 0)),
            pl.BlockSpec((k, n), lambda i: (0, 0)),
            pl.BlockSpec((1, n), lambda i: (0, 0)),
        ],
        out_specs=pl.BlockSpec((block, n), lambda i: (i, 0)),
        out_shape=jax.ShapeDtypeStruct((m, n), jnp.float32),
    )(x, w, b)


def _lstm_step(pre_ref, r, t, h, c):
    pf = pre_ref[pl.ds(t, 1), 0:256]
    pb = pre_ref[pl.ds(N - 1 - t, 1), 256:512]
    g = jnp.concatenate([pf, pb], axis=1) + _dot(h, r)
    sg = jax.nn.sigmoid(g)
    th = jnp.tanh(g)
    i_cat = jnp.concatenate([sg[:, 0:64], sg[:, 256:320]], axis=1)
    f_cat = jnp.concatenate([sg[:, 64:128], sg[:, 320:384]], axis=1)
    g_cat = jnp.concatenate([th[:, 128:192], th[:, 384:448]], axis=1)
    o_cat = jnp.concatenate([sg[:, 192:256], sg[:, 448:512]], axis=1)
    c = f_cat * c + i_cat * g_cat
    h = o_cat * jnp.tanh(c)
    return h, c


def _lstm1_kernel(pre1_ref, r1_ref, out1_ref):
    r1 = r1_ref[...]

    def step1(t, carry):
        h, c = carry
        h, c = _lstm_step(pre1_ref, r1, t, h, c)
        out1_ref[pl.ds(t, 1), 0:64] = h[:, 0:64]
        out1_ref[pl.ds(N - 1 - t, 1), 64:128] = h[:, 64:128]
        return h, c

    z = jnp.zeros((1, 128), jnp.float32)
    jax.lax.fori_loop(0, N, step1, (z, z))


def _lstm2_heads_kernel(pre2_ref, r2_ref, hw_ref, hb_ref, out_ref):
    r2 = r2_ref[...]

    def step2(t, carry):
        h, c, acc = carry
        h, c = _lstm_step(pre2_ref, r2, t, h, c)
        return h, c, acc + h

    z = jnp.zeros((1, 128), jnp.float32)
    _, _, s = jax.lax.fori_loop(0, N, step2, (z, z, z))

    xm = s * (1.0 / N)

    emb = _dot(xm, hw_ref[:, 0:32]) + hb_ref[0:1, 0:32]
    c1 = hw_ref[0:32, 32:96]
    a1 = hw_ref[32:64, 32:96]
    c2 = hw_ref[64:128, 96:98]
    a2 = hw_ref[64:128, 98:99]
    hc = jnp.maximum(_dot(emb, c1) + hb_ref[1:2, 0:64], 0.0)
    logits = _dot(hc, c2) + hb_ref[3:4, 0:2]
    ha = jnp.maximum(_dot(emb, a1) + hb_ref[2:3, 0:64], 0.0)
    anom = _dot(ha, a2) + hb_ref[3:4, 2:3]

    out_ref[...] = jnp.zeros((8, 128), jnp.float32)
    out_ref[0:1, 0:32] = emb
    out_ref[1:2, 0:2] = logits
    out_ref[2:3, 0:1] = anom


def _lstm_heads(x, p):
    def cat2(a, b):
        return jnp.concatenate([a, b], axis=1)

    def blockdiag(a, b):
        z = jnp.zeros((64, 256), jnp.float32)
        return jnp.concatenate(
            [jnp.concatenate([a, z], axis=1), jnp.concatenate([z, b], axis=1)],
            axis=0,
        )

    a1c = cat2(p['lstm_l0f_Wih'].T, p['lstm_l0b_Wih'].T)
    r1 = blockdiag(p['lstm_l0f_Whh'].T, p['lstm_l0b_Whh'].T)
    b1 = cat2(
        (p['lstm_l0f_bih'] + p['lstm_l0f_bhh'])[None, :],
        (p['lstm_l0b_bih'] + p['lstm_l0b_bhh'])[None, :],
    )
    a2c = cat2(p['lstm_l1f_Wih'].T, p['lstm_l1b_Wih'].T)
    r2 = blockdiag(p['lstm_l1f_Whh'].T, p['lstm_l1b_Whh'].T)
    b2 = cat2(
        (p['lstm_l1f_bih'] + p['lstm_l1f_bhh'])[None, :],
        (p['lstm_l1b_bih'] + p['lstm_l1b_bhh'])[None, :],
    )

    hw = jnp.zeros((128, 128), jnp.float32)
    hw = hw.at[0:128, 0:32].set(p['emb_W'])
    hw = hw.at[0:32, 32:96].set(p['c1_W'])
    hw = hw.at[32:64, 32:96].set(p['a1_W'])
    hw = hw.at[64:128, 96:98].set(p['c2_W'])
    hw = hw.at[64:128, 98:99].set(p['a2_W'])
    hb = jnp.zeros((8, 128), jnp.float32)
    hb = hb.at[0, 0:32].set(p['emb_b'])
    hb = hb.at[1, 0:64].set(p['c1_b'])
    hb = hb.at[2, 0:64].set(p['a1_b'])
    hb = hb.at[3, 0:2].set(p['c2_b'])
    hb = hb.at[3, 2:3].set(p['a2_b'])

    pre1 = _matmul_bias(x, a1c, b1)
    out1 = pl.pallas_call(
        _lstm1_kernel,
        out_shape=jax.ShapeDtypeStruct((N, 128), jnp.float32),
    )(pre1, r1)
    pre2 = _matmul_bias(out1, a2c, b2)
    out = pl.pallas_call(
        _lstm2_heads_kernel,
        out_shape=jax.ShapeDtypeStruct((8, 128), jnp.float32),
    )(pre2, r2, hw, hb)

    emb = out[0, 0:32]
    logits = out[1, 0:2]
    anom = out[2, 0:1]
    return logits, anom, emb


def _gcn_t(x, src, dst, W, b, n):
    h = x @ W
    loop = jnp.arange(n)
    s = jnp.concatenate([src, loop])
    d = jnp.concatenate([dst, loop])
    ones = jnp.ones(s.shape[0], dtype=h.dtype)
    deg = jax.ops.segment_sum(ones, d, num_segments=n)
    dinv = jax.lax.rsqrt(jnp.maximum(deg, 1.0))
    norm = dinv[s] * dinv[d]
    return jax.ops.segment_sum(h[s] * norm[:, None], d, num_segments=n) + b


def _gat_t(x, src, dst, p, n, heads=2):
    h = (x @ p['gat_W']).reshape(n, heads, -1)
    loop = jnp.arange(n)
    s = jnp.concatenate([src, loop])
    d = jnp.concatenate([dst, loop])
    a_s = jnp.einsum('nhc,hc->nh', h, p['gat_a_src'])
    a_d = jnp.einsum('nhc,hc->nh', h, p['gat_a_dst'])
    e = jax.nn.leaky_relu(a_s[s] + a_d[d], 0.2)
    emax = jax.ops.segment_max(e, d, num_segments=n)
    emax = jnp.where(jnp.isfinite(emax), emax, 0.0)
    w = jnp.exp(e - emax[d])
    denom = jax.ops.segment_sum(w, d, num_segments=n)
    attn = w / (denom[d] + 1e-16)
    out = jax.ops.segment_sum(h[s] * attn[:, :, None], d, num_segments=n)
    return out.mean(axis=1) + p['gat_b']


def _graph_part(x, edge_index, p):
    n = x.shape[0]
    src, dst = edge_index[0], edge_index[1]
    h = jax.nn.relu(_gcn_t(x, src, dst, p['gcn1_W'], p['gcn1_b'], n))
    h = jax.nn.relu(_gcn_t(h, src, dst, p['gcn2_W'], p['gcn2_b'], n))
    return _gat_t(h, src, dst, p, n)


def kernel(x, edge_index, params):
    h = _graph_part(x, edge_index, params)
    return _lstm_heads(h, params)


# speedup vs baseline: 1.0104x; 1.0104x over previous
import functools

import jax
import jax.numpy as jnp
from jax.experimental import pallas as pl
from jax.experimental.pallas import tpu as pltpu

N = 10000
H = 64
D_IN = 128
EMB = 32

_HIGH = jax.lax.Precision.HIGHEST


def _dot(a, b):
    return jax.lax.dot(a, b, precision=_HIGH, preferred_element_type=jnp.float32)


def _matmul_bias_kernel(x_ref, w_ref, b_ref, o_ref):
    o_ref[...] = _dot(x_ref[...], w_ref[...]) + b_ref[...]


def _matmul_bias(x, w, b, block=1000):
    m, k = x.shape
    n = w.shape[1]
    return pl.pallas_call(
        _matmul_bias_kernel,
        grid=(m // block,),
        in_specs=[
            pl.BlockSpec((block, k), lambda i: (i, 0)),
            pl.BlockSpec((k, n), lambda i: (0, 0)),
            pl.BlockSpec((1, n), lambda i: (0, 0)),
        ],
        out_specs=pl.BlockSpec((block, n), lambda i: (i, 0)),
        out_shape=jax.ShapeDtypeStruct((m, n), jnp.float32),
    )(x, w, b)


_REC_PREC = jax.lax.Precision.DEFAULT


def _rdot(a, b):
    return jax.lax.dot(a, b, precision=_REC_PREC,
                       preferred_element_type=jnp.float32)


def _cell(g, c):
    i = jax.nn.sigmoid(g[:, 0:64])
    f = jax.nn.sigmoid(g[:, 64:128])
    gg = jnp.tanh(g[:, 128:192])
    o = jax.nn.sigmoid(g[:, 192:256])
    c = f * c + i * gg
    h = o * jnp.tanh(c)
    return h, c


def _lstm1_kernel(pre1_ref, rf_ref, rb_ref, out1_ref):
    rf = rf_ref[...]
    rb = rb_ref[...]

    def step1(t, carry):
        hf, cf, hb, cb = carry
        gf = pre1_ref[pl.ds(t, 1), 0:256] + _rdot(hf, rf)
        gb = pre1_ref[pl.ds(N - 1 - t, 1), 256:512] + _rdot(hb, rb)
        hf, cf = _cell(gf, cf)
        hb, cb = _cell(gb, cb)
        out1_ref[pl.ds(t, 1), 0:64] = hf
        out1_ref[pl.ds(N - 1 - t, 1), 64:128] = hb
        return hf, cf, hb, cb

    z = jnp.zeros((1, 64), jnp.float32)
    jax.lax.fori_loop(0, N, step1, (z, z, z, z))


def _lstm2_heads_kernel(pre2_ref, rf_ref, rb_ref, hw_ref, hb_ref, out_ref):
    rf = rf_ref[...]
    rb = rb_ref[...]

    def step2(t, carry):
        hf, cf, hb, cb, af, ab = carry
        gf = pre2_ref[pl.ds(t, 1), 0:256] + _rdot(hf, rf)
        gb = pre2_ref[pl.ds(N - 1 - t, 1), 256:512] + _rdot(hb, rb)
        hf, cf = _cell(gf, cf)
        hb, cb = _cell(gb, cb)
        return hf, cf, hb, cb, af + hf, ab + hb

    z = jnp.zeros((1, 64), jnp.float32)
    _, _, _, _, sf, sb = jax.lax.fori_loop(0, N, step2, (z, z, z, z, z, z))

    xm = jnp.concatenate([sf, sb], axis=1) * (1.0 / N)

    emb = _dot(xm, hw_ref[:, 0:32]) + hb_ref[0:1, 0:32]
    c1 = hw_ref[0:32, 32:96]
    a1 = hw_ref[32:64, 32:96]
    c2 = hw_ref[64:128, 96:98]
    a2 = hw_ref[64:128, 98:99]
    hc = jnp.maximum(_dot(emb, c1) + hb_ref[1:2, 0:64], 0.0)
    logits = _dot(hc, c2) + hb_ref[3:4, 0:2]
    ha = jnp.maximum(_dot(emb, a1) + hb_ref[2:3, 0:64], 0.0)
    anom = _dot(ha, a2) + hb_ref[3:4, 2:3]

    out_ref[...] = jnp.zeros((8, 128), jnp.float32)
    out_ref[0:1, 0:32] = emb
    out_ref[1:2, 0:2] = logits
    out_ref[2:3, 0:1] = anom


def _lstm_heads(x, p):
    def cat2(a, b):
        return jnp.concatenate([a, b], axis=1)

    def blockdiag(a, b):
        z = jnp.zeros((64, 256), jnp.float32)
        return jnp.concatenate(
            [jnp.concatenate([a, z], axis=1), jnp.concatenate([z, b], axis=1)],
            axis=0,
        )

    a1c = cat2(p['lstm_l0f_Wih'].T, p['lstm_l0b_Wih'].T)
    rf1 = p['lstm_l0f_Whh'].T
    rb1 = p['lstm_l0b_Whh'].T
    b1 = cat2(
        (p['lstm_l0f_bih'] + p['lstm_l0f_bhh'])[None, :],
        (p['lstm_l0b_bih'] + p['lstm_l0b_bhh'])[None, :],
    )
    a2c = cat2(p['lstm_l1f_Wih'].T, p['lstm_l1b_Wih'].T)
    rf2 = p['lstm_l1f_Whh'].T
    rb2 = p['lstm_l1b_Whh'].T
    b2 = cat2(
        (p['lstm_l1f_bih'] + p['lstm_l1f_bhh'])[None, :],
        (p['lstm_l1b_bih'] + p['lstm_l1b_bhh'])[None, :],
    )

    hw = jnp.zeros((128, 128), jnp.float32)
    hw = hw.at[0:128, 0:32].set(p['emb_W'])
    hw = hw.at[0:32, 32:96].set(p['c1_W'])
    hw = hw.at[32:64, 32:96].set(p['a1_W'])
    hw = hw.at[64:128, 96:98].set(p['c2_W'])
    hw = hw.at[64:128, 98:99].set(p['a2_W'])
    hb = jnp.zeros((8, 128), jnp.float32)
    hb = hb.at[0, 0:32].set(p['emb_b'])
    hb = hb.at[1, 0:64].set(p['c1_b'])
    hb = hb.at[2, 0:64].set(p['a1_b'])
    hb = hb.at[3, 0:2].set(p['c2_b'])
    hb = hb.at[3, 2:3].set(p['a2_b'])

    pre1 = _matmul_bias(x, a1c, b1)
    out1 = pl.pallas_call(
        _lstm1_kernel,
        out_shape=jax.ShapeDtypeStruct((N, 128), jnp.float32),
    )(pre1, rf1, rb1)
    pre2 = _matmul_bias(out1, a2c, b2)
    out = pl.pallas_call(
        _lstm2_heads_kernel,
        out_shape=jax.ShapeDtypeStruct((8, 128), jnp.float32),
    )(pre2, rf2, rb2, hw, hb)

    emb = out[0, 0:32]
    logits = out[1, 0:2]
    anom = out[2, 0:1]
    return logits, anom, emb


def _gcn_t(x, src, dst, W, b, n):
    h = x @ W
    loop = jnp.arange(n)
    s = jnp.concatenate([src, loop])
    d = jnp.concatenate([dst, loop])
    ones = jnp.ones(s.shape[0], dtype=h.dtype)
    deg = jax.ops.segment_sum(ones, d, num_segments=n)
    dinv = jax.lax.rsqrt(jnp.maximum(deg, 1.0))
    norm = dinv[s] * dinv[d]
    return jax.ops.segment_sum(h[s] * norm[:, None], d, num_segments=n) + b


def _gat_t(x, src, dst, p, n, heads=2):
    h = (x @ p['gat_W']).reshape(n, heads, -1)
    loop = jnp.arange(n)
    s = jnp.concatenate([src, loop])
    d = jnp.concatenate([dst, loop])
    a_s = jnp.einsum('nhc,hc->nh', h, p['gat_a_src'])
    a_d = jnp.einsum('nhc,hc->nh', h, p['gat_a_dst'])
    e = jax.nn.leaky_relu(a_s[s] + a_d[d], 0.2)
    emax = jax.ops.segment_max(e, d, num_segments=n)
    emax = jnp.where(jnp.isfinite(emax), emax, 0.0)
    w = jnp.exp(e - emax[d])
    denom = jax.ops.segment_sum(w, d, num_segments=n)
    attn = w / (denom[d] + 1e-16)
    out = jax.ops.segment_sum(h[s] * attn[:, :, None], d, num_segments=n)
    return out.mean(axis=1) + p['gat_b']


def _graph_part(x, edge_index, p):
    n = x.shape[0]
    src, dst = edge_index[0], edge_index[1]
    h = jax.nn.relu(_gcn_t(x, src, dst, p['gcn1_W'], p['gcn1_b'], n))
    h = jax.nn.relu(_gcn_t(h, src, dst, p['gcn2_W'], p['gcn2_b'], n))
    return _gat_t(h, src, dst, p, n)


def kernel(x, edge_index, params):
    h = _graph_part(x, edge_index, params)
    return _lstm_heads(h, params)


# speedup vs baseline: 1.0178x; 1.0073x over previous
import functools

import jax
import jax.numpy as jnp
from jax.experimental import pallas as pl
from jax.experimental.pallas import tpu as pltpu

N = 10000
H = 64
D_IN = 128
EMB = 32

_HIGH = jax.lax.Precision.HIGHEST


def _dot(a, b):
    return jax.lax.dot(a, b, precision=_HIGH, preferred_element_type=jnp.float32)


def _matmul_bias_kernel(x_ref, w_ref, b_ref, o_ref):
    o_ref[...] = _dot(x_ref[...], w_ref[...]) + b_ref[...]


def _matmul_bias(x, w, b, block=1000):
    m, k = x.shape
    n = w.shape[1]
    return pl.pallas_call(
        _matmul_bias_kernel,
        grid=(m // block,),
        in_specs=[
            pl.BlockSpec((block, k), lambda i: (i, 0)),
            pl.BlockSpec((k, n), lambda i: (0, 0)),
            pl.BlockSpec((1, n), lambda i: (0, 0)),
        ],
        out_specs=pl.BlockSpec((block, n), lambda i: (i, 0)),
        out_shape=jax.ShapeDtypeStruct((m, n), jnp.float32),
    )(x, w, b)


def _bdot(a, b):
    return jax.lax.dot(a, b, preferred_element_type=jnp.float32)


def _rdot3(h, wh, wl):
    hh = h.astype(jnp.bfloat16)
    hl = (h - hh.astype(jnp.float32)).astype(jnp.bfloat16)
    return _bdot(hh, wh) + (_bdot(hh, wl) + _bdot(hl, wh))


def _cell(g, c):
    i = jax.nn.sigmoid(g[:, 0:64])
    f = jax.nn.sigmoid(g[:, 64:128])
    gg = jnp.tanh(g[:, 128:192])
    o = jax.nn.sigmoid(g[:, 192:256])
    c = f * c + i * gg
    h = o * jnp.tanh(c)
    return h, c


_UNROLL = 8


def _lstm1_kernel(pre1_ref, rfh_ref, rfl_ref, rbh_ref, rbl_ref, out1_ref):
    rfh, rfl = rfh_ref[...], rfl_ref[...]
    rbh, rbl = rbh_ref[...], rbl_ref[...]

    def step1(j, carry):
        hf, cf, hb, cb = carry
        base_f = j * _UNROLL
        base_b = N - _UNROLL - j * _UNROLL
        pf = pre1_ref[pl.ds(base_f, _UNROLL), 0:256]
        pb = pre1_ref[pl.ds(base_b, _UNROLL), 256:512]
        hfs, hbs = [], []
        for i in range(_UNROLL):
            gf = pf[i:i + 1, :] + _rdot3(hf, rfh, rfl)
            gb = pb[_UNROLL - 1 - i:_UNROLL - i, :] + _rdot3(hb, rbh, rbl)
            hf, cf = _cell(gf, cf)
            hb, cb = _cell(gb, cb)
            hfs.append(hf)
            hbs.append(hb)
        out1_ref[pl.ds(base_f, _UNROLL), 0:64] = jnp.concatenate(hfs, axis=0)
        out1_ref[pl.ds(base_b, _UNROLL), 64:128] = jnp.concatenate(
            hbs[::-1], axis=0)
        return hf, cf, hb, cb

    z = jnp.zeros((1, 64), jnp.float32)
    jax.lax.fori_loop(0, N // _UNROLL, step1, (z, z, z, z))


def _lstm2_heads_kernel(pre2_ref, rfh_ref, rfl_ref, rbh_ref, rbl_ref,
                        hw_ref, hb_ref, out_ref):
    rfh, rfl = rfh_ref[...], rfl_ref[...]
    rbh, rbl = rbh_ref[...], rbl_ref[...]

    def step2(j, carry):
        hf, cf, hb, cb, af, ab = carry
        base_f = j * _UNROLL
        base_b = N - _UNROLL - j * _UNROLL
        pf = pre2_ref[pl.ds(base_f, _UNROLL), 0:256]
        pb = pre2_ref[pl.ds(base_b, _UNROLL), 256:512]
        for i in range(_UNROLL):
            gf = pf[i:i + 1, :] + _rdot3(hf, rfh, rfl)
            gb = pb[_UNROLL - 1 - i:_UNROLL - i, :] + _rdot3(hb, rbh, rbl)
            hf, cf = _cell(gf, cf)
            hb, cb = _cell(gb, cb)
            af = af + hf
            ab = ab + hb
        return hf, cf, hb, cb, af, ab

    z = jnp.zeros((1, 64), jnp.float32)
    _, _, _, _, sf, sb = jax.lax.fori_loop(
        0, N // _UNROLL, step2, (z, z, z, z, z, z))

    xm = jnp.concatenate([sf, sb], axis=1) * (1.0 / N)

    emb = _dot(xm, hw_ref[:, 0:32]) + hb_ref[0:1, 0:32]
    c1 = hw_ref[0:32, 32:96]
    a1 = hw_ref[32:64, 32:96]
    c2 = hw_ref[64:128, 96:98]
    a2 = hw_ref[64:128, 98:99]
    hc = jnp.maximum(_dot(emb, c1) + hb_ref[1:2, 0:64], 0.0)
    logits = _dot(hc, c2) + hb_ref[3:4, 0:2]
    ha = jnp.maximum(_dot(emb, a1) + hb_ref[2:3, 0:64], 0.0)
    anom = _dot(ha, a2) + hb_ref[3:4, 2:3]

    out_ref[...] = jnp.zeros((8, 128), jnp.float32)
    out_ref[0:1, 0:32] = emb
    out_ref[1:2, 0:2] = logits
    out_ref[2:3, 0:1] = anom


def _lstm_heads(x, p):
    def cat2(a, b):
        return jnp.concatenate([a, b], axis=1)

    def blockdiag(a, b):
        z = jnp.zeros((64, 256), jnp.float32)
        return jnp.concatenate(
            [jnp.concatenate([a, z], axis=1), jnp.concatenate([z, b], axis=1)],
            axis=0,
        )

    a1c = cat2(p['lstm_l0f_Wih'].T, p['lstm_l0b_Wih'].T)
    rf1 = p['lstm_l0f_Whh'].T
    rb1 = p['lstm_l0b_Whh'].T
    b1 = cat2(
        (p['lstm_l0f_bih'] + p['lstm_l0f_bhh'])[None, :],
        (p['lstm_l0b_bih'] + p['lstm_l0b_bhh'])[None, :],
    )
    a2c = cat2(p['lstm_l1f_Wih'].T, p['lstm_l1b_Wih'].T)
    rf2 = p['lstm_l1f_Whh'].T
    rb2 = p['lstm_l1b_Whh'].T
    b2 = cat2(
        (p['lstm_l1f_bih'] + p['lstm_l1f_bhh'])[None, :],
        (p['lstm_l1b_bih'] + p['lstm_l1b_bhh'])[None, :],
    )

    hw = jnp.zeros((128, 128), jnp.float32)
    hw = hw.at[0:128, 0:32].set(p['emb_W'])
    hw = hw.at[0:32, 32:96].set(p['c1_W'])
    hw = hw.at[32:64, 32:96].set(p['a1_W'])
    hw = hw.at[64:128, 96:98].set(p['c2_W'])
    hw = hw.at[64:128, 98:99].set(p['a2_W'])
    hb = jnp.zeros((8, 128), jnp.float32)
    hb = hb.at[0, 0:32].set(p['emb_b'])
    hb = hb.at[1, 0:64].set(p['c1_b'])
    hb = hb.at[2, 0:64].set(p['a1_b'])
    hb = hb.at[3, 0:2].set(p['c2_b'])
    hb = hb.at[3, 2:3].set(p['a2_b'])

    def hilo(w):
        wh = w.astype(jnp.bfloat16)
        wl = (w - wh.astype(jnp.float32)).astype(jnp.bfloat16)
        return wh, wl

    rf1h, rf1l = hilo(rf1)
    rb1h, rb1l = hilo(rb1)
    rf2h, rf2l = hilo(rf2)
    rb2h, rb2l = hilo(rb2)

    pre1 = _matmul_bias(x, a1c, b1)
    out1 = pl.pallas_call(
        _lstm1_kernel,
        out_shape=jax.ShapeDtypeStruct((N, 128), jnp.float32),
    )(pre1, rf1h, rf1l, rb1h, rb1l)
    pre2 = _matmul_bias(out1, a2c, b2)
    out = pl.pallas_call(
        _lstm2_heads_kernel,
        out_shape=jax.ShapeDtypeStruct((8, 128), jnp.float32),
    )(pre2, rf2h, rf2l, rb2h, rb2l, hw, hb)

    emb = out[0, 0:32]
    logits = out[1, 0:2]
    anom = out[2, 0:1]
    return logits, anom, emb


def _gcn_t(x, src, dst, W, b, n):
    h = x @ W
    loop = jnp.arange(n)
    s = jnp.concatenate([src, loop])
    d = jnp.concatenate([dst, loop])
    ones = jnp.ones(s.shape[0], dtype=h.dtype)
    deg = jax.ops.segment_sum(ones, d, num_segments=n)
    dinv = jax.lax.rsqrt(jnp.maximum(deg, 1.0))
    norm = dinv[s] * dinv[d]
    return jax.ops.segment_sum(h[s] * norm[:, None], d, num_segments=n) + b


def _gat_t(x, src, dst, p, n, heads=2):
    h = (x @ p['gat_W']).reshape(n, heads, -1)
    loop = jnp.arange(n)
    s = jnp.concatenate([src, loop])
    d = jnp.concatenate([dst, loop])
    a_s = jnp.einsum('nhc,hc->nh', h, p['gat_a_src'])
    a_d = jnp.einsum('nhc,hc->nh', h, p['gat_a_dst'])
    e = jax.nn.leaky_relu(a_s[s] + a_d[d], 0.2)
    emax = jax.ops.segment_max(e, d, num_segments=n)
    emax = jnp.where(jnp.isfinite(emax), emax, 0.0)
    w = jnp.exp(e - emax[d])
    denom = jax.ops.segment_sum(w, d, num_segments=n)
    attn = w / (denom[d] + 1e-16)
    out = jax.ops.segment_sum(h[s] * attn[:, :, None], d, num_segments=n)
    return out.mean(axis=1) + p['gat_b']


def _graph_part(x, edge_index, p):
    n = x.shape[0]
    src, dst = edge_index[0], edge_index[1]
    h = jax.nn.relu(_gcn_t(x, src, dst, p['gcn1_W'], p['gcn1_b'], n))
    h = jax.nn.relu(_gcn_t(h, src, dst, p['gcn2_W'], p['gcn2_b'], n))
    return _gat_t(h, src, dst, p, n)


def kernel(x, edge_index, params):
    h = _graph_part(x, edge_index, params)
    return _lstm_heads(h, params)
